# baseline (speedup 1.0000x reference)
import functools

import jax
import jax.numpy as jnp
from jax import lax
from jax.experimental import pallas as pl
from jax.experimental.pallas import tpu as pltpu
from jax.experimental.pallas import tpu_sc as plsc

_L = 200
_B = 4096
_D = 64
_TB = 128
_HU = 2
_TU = _TB * _HU
_NG = 4


def _build_kernel(V):
    info = plsc.get_sparse_core_info()
    NC, NS = info.num_cores, info.num_subcores
    NW = NC * NS
    NTC = _B // _TB
    NU = _B // _TU
    UNITS = _L * NU
    per_w = UNITS // NW
    assert UNITS % NW == 0 and per_w % _NG == 0

    mesh = plsc.VectorSubcoreMesh(core_axis_name="c", subcore_axis_name="s")

    @functools.partial(
        pl.kernel,
        mesh=mesh,
        out_type=jax.ShapeDtypeStruct((_L, _D // 8, NTC, 8, _TB), jnp.float32),
        compiler_params=pltpu.CompilerParams(
            use_tc_tiling_on_sc=False, needs_layout_passes=False),
        scratch_types=(
            [pltpu.VMEM((_TU,), jnp.int32) for _ in range(_NG)]
            + [pltpu.VMEM((_TU, _D), jnp.float32) for _ in range(_NG)]
            + [pltpu.VMEM((_D // 8, 8, _TB + 1), jnp.float32)
               for _ in range(2 * _HU)]
            + [pltpu.VMEM((_L * _D,), jnp.float32)]
            + [pltpu.SemaphoreType.DMA for _ in range(2 * _NG + 2)]
        ),
    )
    def k(idx_hbm, tab_hbm, pos_hbm, out_hbm, *scr):
        idx_bufs = scr[:_NG]
        rows_bufs = scr[_NG:2 * _NG]
        tr_bufs = scr[2 * _NG:2 * _NG + 2 * _HU]
        pos_v = scr[2 * _NG + 2 * _HU]
        isems = scr[2 * _NG + 2 * _HU + 1:3 * _NG + 2 * _HU + 1]
        gsems = scr[3 * _NG + 2 * _HU + 1:4 * _NG + 2 * _HU + 1]
        ssems = scr[4 * _NG + 2 * _HU + 1:]

        wid = lax.axis_index("s") * NC + lax.axis_index("c")
        u_base = wid * per_w

        pltpu.sync_copy(pos_hbm, pos_v)

        iota16 = lax.iota(jnp.int32, 16)
        tvecs = [(iota16 + 16 * t) // 8 for t in range(_D // 16)]
        svecs = [(iota16 + 16 * t) % 8 for t in range(_D // 16)]

        def unit_lc(u):
            U = u_base + u
            return U // NU, (U % NU) * _HU

        def start_idx(u, b):
            l, tc = unit_lc(u)
            return pltpu.async_copy(
                idx_hbm.at[pl.ds(l * _B + tc * _TB, _TU)], idx_bufs[b],
                isems[b])

        def wait_idx(u, b):
            l, tc = unit_lc(u)
            pltpu.make_async_copy(
                idx_hbm.at[pl.ds(l * _B + tc * _TB, _TU)], idx_bufs[b],
                isems[b]).wait()

        def start_gather(b):
            for h in range(_HU):
                pltpu.async_copy(
                    tab_hbm.at[idx_bufs[b].at[pl.ds(h * _TB, _TB)]],
                    rows_bufs[b].at[pl.ds(h * _TB, _TB)], gsems[b])

        def wait_gather(b):
            for h in range(_HU):
                pltpu.make_async_copy(
                    tab_hbm.at[idx_bufs[b].at[pl.ds(h * _TB, _TB)]],
                    rows_bufs[b].at[pl.ds(h * _TB, _TB)], gsems[b]).wait()

        def do_transpose_add(l, rows, tb):
            pos_base = l * _D
            pvs = [pos_v[pl.ds(pos_base + 16 * t, 16)] for t in range(_D // 16)]

            for h in range(_HU):
                trn = tr_bufs[2 * tb + h]

                def body(q, _):
                    for j in range(4):
                        r = q * 4 + j
                        rvec = jnp.full((16,), r, jnp.int32)
                        for t in range(_D // 16):
                            v = rows[h * _TB + r, pl.ds(16 * t, 16)] + pvs[t]
                            plsc.store_scatter(
                                trn, [tvecs[t], svecs[t], rvec], v)
                    return 0

                lax.fori_loop(0, _TB // 4, body, 0)

        def start_scatter(u, tb):
            l, tc = unit_lc(u)
            for h in range(_HU):
                pltpu.async_copy(
                    tr_bufs[2 * tb + h].at[:, :, pl.ds(0, _TB)],
                    out_hbm.at[l, :, tc + h], ssems[tb])

        def wait_scatter(u, tb):
            l, tc = unit_lc(u)
            for h in range(_HU):
                pltpu.make_async_copy(
                    tr_bufs[2 * tb + h].at[:, :, pl.ds(0, _TB)],
                    out_hbm.at[l, :, tc + h], ssems[tb]).wait()

        for b in range(_NG):
            start_idx(b, b)
        for b in range(_NG - 1):
            wait_idx(b, b)
            start_gather(b)

        def group(p, _):
            for b in range(_NG):
                u = _NG * p + b
                bg = (b + _NG - 1) % _NG
                tb = b % 2

                @pl.when(u + _NG - 1 < per_w)
                def _():
                    wait_idx(u + _NG - 1, bg)
                    start_gather(bg)

                wait_gather(b)

                @pl.when(u + _NG < per_w)
                def _():
                    start_idx(u + _NG, b)

                @pl.when(u >= 2)
                def _():
                    wait_scatter(u - 2, tb)

                l, _tc = unit_lc(u)
                do_transpose_add(l, rows_bufs[b], tb)
                start_scatter(u, tb)
            return 0

        lax.fori_loop(0, per_w // _NG, group, 0)
        wait_scatter(per_w - 2, 0)
        wait_scatter(per_w - 1, 1)

    return k


def kernel(inputs, token_table, pos_table):
    B, L = inputs.shape
    V, D = token_table.shape
    idx_t = jnp.transpose(inputs).reshape(L * B).astype(jnp.int32)
    pos_flat = pos_table.reshape(L * D)
    k = _build_kernel(V)
    out5 = k(idx_t, token_table, pos_flat)
    return out5.transpose(2, 4, 0, 1, 3).reshape(B, L, D)


# speedup vs baseline: 1.0365x; 1.0365x over previous
import functools

import jax
import jax.numpy as jnp
from jax import lax
from jax.experimental import pallas as pl
from jax.experimental.pallas import tpu as pltpu
from jax.experimental.pallas import tpu_sc as plsc

_L = 200
_B = 4096
_D = 64
_TB = 128
_NG = 4


def _build_kernel(V):
    info = plsc.get_sparse_core_info()
    NC, NS = info.num_cores, info.num_subcores
    NW = NC * NS
    NTC = _B // _TB
    UNITS = _L * NTC
    per_w = UNITS // NW
    assert UNITS % NW == 0 and per_w % _NG == 0

    mesh = plsc.VectorSubcoreMesh(core_axis_name="c", subcore_axis_name="s")

    @functools.partial(
        pl.kernel,
        mesh=mesh,
        out_type=jax.ShapeDtypeStruct((_L, _D // 8, NTC, 8, _TB), jnp.float32),
        compiler_params=pltpu.CompilerParams(
            use_tc_tiling_on_sc=False, needs_layout_passes=False),
        scratch_types=(
            [pltpu.VMEM((_TB,), jnp.int32) for _ in range(_NG)]
            + [pltpu.VMEM((_TB, _D), jnp.float32) for _ in range(_NG)]
            + [pltpu.VMEM((_D // 8, 8, _TB + 1), jnp.float32) for _ in range(4)]
            + [pltpu.VMEM((_L * _D,), jnp.float32)]
            + [pltpu.SemaphoreType.DMA for _ in range(2 * _NG + 4)]
        ),
    )
    def k(idx_hbm, tab_hbm, pos_hbm, out_hbm, *scr):
        idx_bufs = scr[:_NG]
        rows_bufs = scr[_NG:2 * _NG]
        tr_bufs = scr[2 * _NG:2 * _NG + 4]
        pos_v = scr[2 * _NG + 4]
        isems = scr[2 * _NG + 5:3 * _NG + 5]
        gsems = scr[3 * _NG + 5:4 * _NG + 5]
        ssems = scr[4 * _NG + 5:]

        wid = lax.axis_index("s") * NC + lax.axis_index("c")
        u_base = wid * per_w

        pltpu.sync_copy(pos_hbm, pos_v)

        iota16 = lax.iota(jnp.int32, 16)
        tvecs = [(iota16 + 16 * t) // 8 for t in range(_D // 16)]
        svecs = [(iota16 + 16 * t) % 8 for t in range(_D // 16)]

        def unit_lc(u):
            U = u_base + u
            return U // NTC, U % NTC

        def start_idx(u, b):
            l, tc = unit_lc(u)
            return pltpu.async_copy(
                idx_hbm.at[pl.ds(l * _B + tc * _TB, _TB)], idx_bufs[b],
                isems[b])

        def wait_idx(u, b):
            l, tc = unit_lc(u)
            pltpu.make_async_copy(
                idx_hbm.at[pl.ds(l * _B + tc * _TB, _TB)], idx_bufs[b],
                isems[b]).wait()

        def start_gather(b):
            return pltpu.async_copy(
                tab_hbm.at[idx_bufs[b]], rows_bufs[b], gsems[b])

        def wait_gather(b):
            pltpu.make_async_copy(
                tab_hbm.at[idx_bufs[b]], rows_bufs[b], gsems[b]).wait()

        def do_transpose_add(l, rows, trn):
            pos_base = l * _D
            pvs = [pos_v[pl.ds(pos_base + 16 * t, 16)] for t in range(_D // 16)]

            def body(q, _):
                for j in range(4):
                    r = q * 4 + j
                    rvec = jnp.full((16,), r, jnp.int32)
                    for t in range(_D // 16):
                        v = rows[r, pl.ds(16 * t, 16)] + pvs[t]
                        plsc.store_scatter(trn, [tvecs[t], svecs[t], rvec], v)
                return 0

            lax.fori_loop(0, _TB // 4, body, 0)

        def start_scatter(u, tb):
            l, tc = unit_lc(u)
            pltpu.async_copy(
                tr_bufs[tb].at[:, :, pl.ds(0, _TB)],
                out_hbm.at[l, :, tc], ssems[tb])

        def wait_scatter(u, tb):
            l, tc = unit_lc(u)
            pltpu.make_async_copy(
                tr_bufs[tb].at[:, :, pl.ds(0, _TB)],
                out_hbm.at[l, :, tc], ssems[tb]).wait()

        for b in range(_NG):
            start_idx(b, b)
        for b in range(_NG - 1):
            wait_idx(b, b)
            start_gather(b)

        def group(p, _):
            for b in range(_NG):
                u = _NG * p + b
                bg = (b + _NG - 1) % _NG
                tb = b

                @pl.when(u + _NG - 1 < per_w)
                def _():
                    wait_idx(u + _NG - 1, bg)
                    start_gather(bg)

                wait_gather(b)

                @pl.when(u + _NG < per_w)
                def _():
                    start_idx(u + _NG, b)

                @pl.when(u >= _NG)
                def _():
                    wait_scatter(u - _NG, tb)

                l, _tc = unit_lc(u)
                do_transpose_add(l, rows_bufs[b], tr_bufs[tb])
                start_scatter(u, tb)
            return 0

        lax.fori_loop(0, per_w // _NG, group, 0)
        for b in range(_NG):
            wait_scatter(per_w - _NG + b, b)

    return k


def kernel(inputs, token_table, pos_table):
    B, L = inputs.shape
    V, D = token_table.shape
    idx_t = jnp.transpose(inputs).reshape(L * B).astype(jnp.int32)
    pos_flat = pos_table.reshape(L * D)
    k = _build_kernel(V)
    out5 = k(idx_t, token_table, pos_flat)
    return out5.transpose(2, 4, 0, 1, 3).reshape(B, L, D)
